# baseline (speedup 1.0000x reference)
import functools

import jax
import jax.numpy as jnp
from jax import lax
from jax.experimental import pallas as pl
from jax.experimental.pallas import tpu as pltpu
from jax.experimental.pallas import tpu_sc as plsc

_DIM = 128
_NUM_COUNT = 10
_NC = 2
_NS = 16
_NW = _NC * _NS
_CHUNK = 128
_S_TC = 12
_TC_BLK = 1024


@functools.lru_cache(maxsize=None)
def _build_sc_gather(n_total: int, n_sc: int):
    assert n_sc % (_NW * _CHUNK) == 0
    chunks_per_w = n_sc // (_NW * _CHUNK)
    b_per_w = n_sc // _NW
    off = n_total - n_sc
    mesh = plsc.VectorSubcoreMesh(core_axis_name="c", subcore_axis_name="s")

    def body(table_hbm, idx_hbm, out_hbm, table_sh, idx_v, rows_v, gsem,
             ssem):
        sid = lax.axis_index("s")
        wid = sid * _NC + lax.axis_index("c")
        base = off + wid * b_per_w
        @pl.when(sid == 0)
        def _():
            pltpu.sync_copy(table_hbm, table_sh)
        pltpu.sync_copy(idx_hbm.at[wid], idx_v)
        plsc.subcore_barrier()
        pltpu.async_copy(table_sh.at[idx_v.at[0]],
                         rows_v.at[pl.ds(0, _CHUNK)], gsem)

        @pl.loop(0, chunks_per_w)
        def _chunk(j):
            @pl.when(j < chunks_per_w - 1)
            def _():
                pltpu.async_copy(
                    table_sh.at[idx_v.at[j + 1]],
                    rows_v.at[pl.ds((j + 1) * _CHUNK, _CHUNK)],
                    gsem,
                )
            pltpu.make_async_copy(
                table_sh.at[idx_v.at[j]],
                rows_v.at[pl.ds(j * _CHUNK, _CHUNK)],
                gsem,
            ).wait()
            pltpu.async_copy(
                rows_v.at[pl.ds(j * _CHUNK, _CHUNK)],
                out_hbm.at[pl.ds(base + j * _CHUNK, _CHUNK)],
                ssem,
            )
        pltpu.make_async_copy(
            rows_v, out_hbm.at[pl.ds(base, b_per_w)], ssem).wait()

    return pl.kernel(
        body,
        out_type=jax.ShapeDtypeStruct((n_total, _DIM), jnp.float32),
        mesh=mesh,
        scratch_types=[
            pltpu.VMEM_SHARED((128, _DIM), jnp.float32),
            pltpu.VMEM((chunks_per_w, _CHUNK), jnp.int32),
            pltpu.VMEM((b_per_w, _DIM), jnp.float32),
            pltpu.SemaphoreType.DMA,
            pltpu.SemaphoreType.DMA,
        ],
    )


def _tc_body(tok_ref, w_ref, _, out_ref):
    tok = tok_ref[...]
    r = _TC_BLK // _DIM
    k = lax.broadcasted_iota(jnp.int32, (r, _DIM, _DIM), 2)
    onehot = (tok[:, :, None] == k).astype(jnp.float32)
    out_ref[...] = lax.dot_general(
        onehot.reshape(_TC_BLK, _DIM), w_ref[...],
        (((1,), (0,)), ((), ())),
        precision=lax.Precision.HIGHEST,
        preferred_element_type=jnp.float32,
    )


@functools.lru_cache(maxsize=None)
def _build_tc_fill(n_total: int, n_tc: int):
    assert n_tc % _TC_BLK == 0
    r = _TC_BLK // _DIM
    return pl.pallas_call(
        _tc_body,
        grid=(n_tc // _TC_BLK,),
        in_specs=[
            pl.BlockSpec((r, _DIM), lambda j: (j, 0)),
            pl.BlockSpec((_DIM, _DIM), lambda j: (0, 0)),
            pl.BlockSpec(memory_space=pl.ANY),
        ],
        out_specs=pl.BlockSpec((_TC_BLK, _DIM), lambda j: (j, 0)),
        out_shape=jax.ShapeDtypeStruct((n_total, _DIM), jnp.float32),
        input_output_aliases={2: 0},
    )


def kernel(x, weight):
    B, S = x.shape
    n = B * S
    n_tc = _S_TC * B
    n_sc = n - n_tc
    dims = jnp.arange(_DIM, dtype=jnp.float32) + 1.0
    num_vals = jnp.arange(_NUM_COUNT, dtype=jnp.float32) / 1000.0
    sinpad = jnp.sin(num_vals[:, None] * dims[None, :])
    table = jnp.concatenate([sinpad, weight[_NUM_COUNT:128]], axis=0)
    xt = x.T
    idx_sc = xt[_S_TC:].reshape(_NW, n_sc // (_NW * _CHUNK), _CHUNK)
    tok_tc = xt[:_S_TC].reshape(n_tc // _DIM, _DIM)
    part = _build_sc_gather(n, n_sc)(table, idx_sc)
    out = _build_tc_fill(n, n_tc)(tok_tc, table, part)
    return out.reshape(S, B, _DIM).transpose(1, 0, 2)


# speedup vs baseline: 1.3161x; 1.3161x over previous
import functools

import jax
import jax.numpy as jnp
from jax import lax
from jax.experimental import pallas as pl
from jax.experimental.pallas import tpu as pltpu
from jax.experimental.pallas import tpu_sc as plsc

_DIM = 128
_NUM_COUNT = 10
_NC = 2
_NS = 16
_NW = _NC * _NS
_CHUNK = 128


@functools.lru_cache(maxsize=None)
def _build_sc_gather(n_tokens: int):
    assert n_tokens % (_NW * _CHUNK) == 0
    chunks_per_w = n_tokens // (_NW * _CHUNK)
    b_per_w = n_tokens // _NW
    mesh = plsc.VectorSubcoreMesh(core_axis_name="c", subcore_axis_name="s")

    def body(table_hbm, idx_hbm, out_hbm, table_sh, idx_v, rows_v, gsem,
             ssem):
        sid = lax.axis_index("s")
        wid = sid * _NC + lax.axis_index("c")
        base = wid * b_per_w
        rows_per_tile = 128 // _NS
        pltpu.sync_copy(
            table_hbm.at[pl.ds(sid * rows_per_tile, rows_per_tile)],
            table_sh.at[pl.ds(sid * rows_per_tile, rows_per_tile)])
        pltpu.sync_copy(idx_hbm.at[wid], idx_v)
        plsc.subcore_barrier()
        pltpu.async_copy(table_sh.at[idx_v.at[0]],
                         rows_v.at[pl.ds(0, _CHUNK)], gsem)

        @pl.loop(0, chunks_per_w)
        def _chunk(j):
            @pl.when(j < chunks_per_w - 1)
            def _():
                pltpu.async_copy(
                    table_sh.at[idx_v.at[j + 1]],
                    rows_v.at[pl.ds((j + 1) * _CHUNK, _CHUNK)],
                    gsem,
                )
            pltpu.make_async_copy(
                table_sh.at[idx_v.at[j]],
                rows_v.at[pl.ds(j * _CHUNK, _CHUNK)],
                gsem,
            ).wait()
            pltpu.async_copy(
                rows_v.at[pl.ds(j * _CHUNK, _CHUNK)],
                out_hbm.at[pl.ds(base + j * _CHUNK, _CHUNK)],
                ssem,
            )
        pltpu.make_async_copy(
            rows_v, out_hbm.at[pl.ds(base, b_per_w)], ssem).wait()

    return pl.kernel(
        body,
        out_type=jax.ShapeDtypeStruct((n_tokens, _DIM), jnp.float32),
        mesh=mesh,
        scratch_types=[
            pltpu.VMEM_SHARED((128, _DIM), jnp.float32),
            pltpu.VMEM((chunks_per_w, _CHUNK), jnp.int32),
            pltpu.VMEM((b_per_w, _DIM), jnp.float32),
            pltpu.SemaphoreType.DMA,
            pltpu.SemaphoreType.DMA,
        ],
    )


def kernel(x, weight):
    B, S = x.shape
    n = B * S
    dims = jnp.arange(_DIM, dtype=jnp.float32) + 1.0
    num_vals = jnp.arange(_NUM_COUNT, dtype=jnp.float32) / 1000.0
    sinpad = jnp.sin(num_vals[:, None] * dims[None, :])
    table = jnp.concatenate([sinpad, weight[_NUM_COUNT:128]], axis=0)
    idx = x.T.reshape(_NW, n // (_NW * _CHUNK), _CHUNK)
    out = _build_sc_gather(n)(table, idx)
    return out.reshape(S, B, _DIM).transpose(1, 0, 2)
